# Initial kernel scaffold; baseline (speedup 1.0000x reference)
import jax
import jax.numpy as jnp
from jax.experimental import pallas as pl


def kernel(cos_cached, sin_cached, position_ids):
    raise NotImplementedError("write your pallas kernel here")


# speedup vs baseline: 4.7268x; 4.7268x over previous
import functools

import jax
import jax.numpy as jnp
from jax import lax
from jax.experimental import pallas as pl
from jax.experimental.pallas import tpu as pltpu
from jax.experimental.pallas import tpu_sc as plsc

DIM = 128
NC = 2
NS = 16
NW = NC * NS
B_TOTAL = 4 * 8192
B_PER_W = B_TOTAL // NW
CHUNK = 128
N_CHUNKS = B_PER_W // CHUNK

_mesh = plsc.VectorSubcoreMesh(core_axis_name="c", subcore_axis_name="s")


@functools.partial(
    pl.kernel,
    mesh=_mesh,
    out_type=jax.ShapeDtypeStruct((2, B_TOTAL, DIM), jnp.float32),
    scratch_types=[
        pltpu.VMEM((N_CHUNKS, CHUNK), jnp.int32),
        pltpu.VMEM((CHUNK, DIM), jnp.float32),
        pltpu.SemaphoreType.DMA,
    ],
)
def _rope_gather(cos_hbm, sin_hbm, idx_hbm, out_hbm, idx_v, buf, sem):
    wid = lax.axis_index("s") * NC + lax.axis_index("c")
    base = wid * B_PER_W
    pltpu.sync_copy(idx_hbm.at[wid], idx_v)
    for t, table in enumerate((cos_hbm, sin_hbm)):
        for c in range(N_CHUNKS):
            pltpu.async_copy(table.at[idx_v.at[c]], buf, sem).wait()
            pltpu.sync_copy(buf, out_hbm.at[t, pl.ds(base + c * CHUNK, CHUNK)])


def kernel(cos_cached, sin_cached, position_ids):
    idx = position_ids.reshape(NW, N_CHUNKS, CHUNK)
    out = _rope_gather(cos_cached, sin_cached, idx)
    return out.reshape(2, 4, 8192, DIM)


# speedup vs baseline: 5.7212x; 1.2104x over previous
import functools

import jax
import jax.numpy as jnp
from jax import lax
from jax.experimental import pallas as pl
from jax.experimental.pallas import tpu as pltpu
from jax.experimental.pallas import tpu_sc as plsc

DIM = 128
NC = 2
NS = 16
NW = NC * NS
B_TOTAL = 4 * 8192
B_PER_W = B_TOTAL // NW
CHUNK = 128
N_CHUNKS = B_PER_W // CHUNK

_mesh = plsc.VectorSubcoreMesh(core_axis_name="c", subcore_axis_name="s")


NBUF = 2


@functools.partial(
    pl.kernel,
    mesh=_mesh,
    out_type=jax.ShapeDtypeStruct((2, B_TOTAL, DIM), jnp.float32),
    scratch_types=[
        pltpu.VMEM((N_CHUNKS, CHUNK), jnp.int32),
        *([pltpu.VMEM((CHUNK, DIM), jnp.float32)] * NBUF),
        *([pltpu.SemaphoreType.DMA] * (2 * NBUF)),
    ],
)
def _rope_gather(cos_hbm, sin_hbm, idx_hbm, out_hbm, idx_v, *bufs_and_sems):
    bufs = bufs_and_sems[:NBUF]
    gsems = bufs_and_sems[NBUF:2 * NBUF]
    ssems = bufs_and_sems[2 * NBUF:]
    wid = lax.axis_index("s") * NC + lax.axis_index("c")
    base = wid * B_PER_W
    pltpu.sync_copy(idx_hbm.at[wid], idx_v)
    tables = (cos_hbm, sin_hbm)
    items = [(t, c) for t in range(2) for c in range(N_CHUNKS)]
    n = len(items)
    gd = [None] * NBUF
    sd = [None] * NBUF
    t0, c0 = items[0]
    gd[0] = pltpu.async_copy(tables[t0].at[idx_v.at[c0]], bufs[0], gsems[0])
    for i in range(n):
        b = i % NBUF
        nb = (i + 1) % NBUF
        if i + 1 < n:
            if sd[nb] is not None:
                sd[nb].wait()
            t1, c1 = items[i + 1]
            gd[nb] = pltpu.async_copy(tables[t1].at[idx_v.at[c1]], bufs[nb], gsems[nb])
        gd[b].wait()
        t, c = items[i]
        sd[b] = pltpu.async_copy(bufs[b], out_hbm.at[t, pl.ds(base + c * CHUNK, CHUNK)], ssems[b])
    for b in range(NBUF):
        if sd[b] is not None:
            sd[b].wait()


def kernel(cos_cached, sin_cached, position_ids):
    idx = position_ids.reshape(NW, N_CHUNKS, CHUNK)
    out = _rope_gather(cos_cached, sin_cached, idx)
    return out.reshape(2, 4, 8192, DIM)


# speedup vs baseline: 5.8470x; 1.0220x over previous
import functools

import jax
import jax.numpy as jnp
from jax import lax
from jax.experimental import pallas as pl
from jax.experimental.pallas import tpu as pltpu
from jax.experimental.pallas import tpu_sc as plsc

DIM = 128
NC = 2
NS = 16
NW = NC * NS
B_TOTAL = 4 * 8192
B_PER_W = B_TOTAL // NW
CHUNK = 128
N_CHUNKS = B_PER_W // CHUNK

_mesh = plsc.VectorSubcoreMesh(core_axis_name="c", subcore_axis_name="s")


NBUF = 4


@functools.partial(
    pl.kernel,
    mesh=_mesh,
    out_type=jax.ShapeDtypeStruct((2, B_TOTAL, DIM), jnp.float32),
    scratch_types=[
        pltpu.VMEM((N_CHUNKS, CHUNK), jnp.int32),
        *([pltpu.VMEM((CHUNK, DIM), jnp.float32)] * NBUF),
        *([pltpu.SemaphoreType.DMA] * (2 * NBUF)),
    ],
)
def _rope_gather(cos_hbm, sin_hbm, idx_hbm, out_hbm, idx_v, *bufs_and_sems):
    bufs = bufs_and_sems[:NBUF]
    gsems = bufs_and_sems[NBUF:2 * NBUF]
    ssems = bufs_and_sems[2 * NBUF:]
    wid = lax.axis_index("s") * NC + lax.axis_index("c")
    base = wid * B_PER_W
    pltpu.sync_copy(idx_hbm.at[wid], idx_v)
    tables = (cos_hbm, sin_hbm)
    items = [(t, c) for t in range(2) for c in range(N_CHUNKS)]
    n = len(items)
    gd = [None] * NBUF
    sd = [None] * NBUF
    for j in range(min(NBUF - 1, n)):
        tj, cj = items[j]
        gd[j] = pltpu.async_copy(tables[tj].at[idx_v.at[cj]], bufs[j], gsems[j])
    for i in range(n):
        b = i % NBUF
        j = i + NBUF - 1
        if j < n:
            jb = j % NBUF
            if sd[jb] is not None:
                sd[jb].wait()
            tj, cj = items[j]
            gd[jb] = pltpu.async_copy(tables[tj].at[idx_v.at[cj]], bufs[jb], gsems[jb])
        gd[b].wait()
        t, c = items[i]
        sd[b] = pltpu.async_copy(bufs[b], out_hbm.at[t, pl.ds(base + c * CHUNK, CHUNK)], ssems[b])
    for b in range(NBUF):
        if sd[b] is not None:
            sd[b].wait()


def kernel(cos_cached, sin_cached, position_ids):
    idx = position_ids.reshape(NW, N_CHUNKS, CHUNK)
    out = _rope_gather(cos_cached, sin_cached, idx)
    return out.reshape(2, 4, 8192, DIM)


# speedup vs baseline: 5.9956x; 1.0254x over previous
import functools

import jax
import jax.numpy as jnp
from jax import lax
from jax.experimental import pallas as pl
from jax.experimental.pallas import tpu as pltpu
from jax.experimental.pallas import tpu_sc as plsc

DIM = 128
NC = 2
NS = 16
NW = NC * NS
B_TOTAL = 4 * 8192
B_PER_W = B_TOTAL // NW
CHUNK = 128
N_CHUNKS = B_PER_W // CHUNK

_mesh = plsc.VectorSubcoreMesh(core_axis_name="c", subcore_axis_name="s")


NBUF = 6


@functools.partial(
    pl.kernel,
    mesh=_mesh,
    out_type=jax.ShapeDtypeStruct((2, B_TOTAL, DIM), jnp.float32),
    scratch_types=[
        pltpu.VMEM((N_CHUNKS, CHUNK), jnp.int32),
        *([pltpu.VMEM((CHUNK, DIM), jnp.float32)] * NBUF),
        *([pltpu.SemaphoreType.DMA] * (2 * NBUF)),
    ],
)
def _rope_gather(cos_hbm, sin_hbm, idx_hbm, out_hbm, idx_v, *bufs_and_sems):
    bufs = bufs_and_sems[:NBUF]
    gsems = bufs_and_sems[NBUF:2 * NBUF]
    ssems = bufs_and_sems[2 * NBUF:]
    wid = lax.axis_index("s") * NC + lax.axis_index("c")
    base = wid * B_PER_W
    pltpu.sync_copy(idx_hbm.at[wid // 8, pl.ds((wid % 8) * 8, N_CHUNKS)], idx_v)
    tables = (cos_hbm, sin_hbm)
    items = [(t, c) for t in range(2) for c in range(N_CHUNKS)]
    n = len(items)
    gd = [None] * NBUF
    sd = [None] * NBUF
    for j in range(min(NBUF - 1, n)):
        tj, cj = items[j]
        gd[j] = pltpu.async_copy(tables[tj].at[idx_v.at[cj]], bufs[j], gsems[j])
    for i in range(n):
        b = i % NBUF
        j = i + NBUF - 1
        if j < n:
            jb = j % NBUF
            if sd[jb] is not None:
                sd[jb].wait()
            tj, cj = items[j]
            gd[jb] = pltpu.async_copy(tables[tj].at[idx_v.at[cj]], bufs[jb], gsems[jb])
        gd[b].wait()
        t, c = items[i]
        sd[b] = pltpu.async_copy(bufs[b], out_hbm.at[t, pl.ds(base + c * CHUNK, CHUNK)], ssems[b])
    for b in range(NBUF):
        if sd[b] is not None:
            sd[b].wait()


def kernel(cos_cached, sin_cached, position_ids):
    idx = position_ids.reshape(4, 64, CHUNK)
    out = _rope_gather(cos_cached, sin_cached, idx)
    return out.reshape(2, 4, 8192, DIM)


# speedup vs baseline: 6.0178x; 1.0037x over previous
import functools

import jax
import jax.numpy as jnp
from jax import lax
from jax.experimental import pallas as pl
from jax.experimental.pallas import tpu as pltpu
from jax.experimental.pallas import tpu_sc as plsc

DIM = 128
NC = 2
NS = 16
NW = NC * NS
B_TOTAL = 4 * 8192
B_PER_W = B_TOTAL // NW
CHUNK = 128
N_CHUNKS = B_PER_W // CHUNK

_mesh = plsc.VectorSubcoreMesh(core_axis_name="c", subcore_axis_name="s")


NBUF = 6


@functools.partial(
    pl.kernel,
    mesh=_mesh,
    out_type=jax.ShapeDtypeStruct((2, B_TOTAL, DIM), jnp.float32),
    scratch_types=[
        pltpu.VMEM((N_CHUNKS, CHUNK), jnp.int32),
        *([pltpu.VMEM((CHUNK, DIM), jnp.float32)] * NBUF),
        *([pltpu.SemaphoreType.DMA] * (2 * NBUF)),
    ],
)
def _rope_gather(cos_hbm, sin_hbm, idx_hbm, out_hbm, idx_v, *bufs_and_sems):
    bufs = bufs_and_sems[:NBUF]
    gsems = bufs_and_sems[NBUF:2 * NBUF]
    ssems = bufs_and_sems[2 * NBUF:]
    wid = lax.axis_index("s") * NC + lax.axis_index("c")
    base = wid * B_PER_W
    pltpu.sync_copy(idx_hbm.at[wid // 8, pl.ds((wid % 8) * 8, N_CHUNKS)], idx_v)
    tables = (cos_hbm, sin_hbm)
    items = [(t, c) for c in range(N_CHUNKS) for t in range(2)]
    n = len(items)
    gd = [None] * NBUF
    sd = [None] * NBUF
    for j in range(min(NBUF - 1, n)):
        tj, cj = items[j]
        gd[j] = pltpu.async_copy(tables[tj].at[idx_v.at[cj]], bufs[j], gsems[j])
    for i in range(n):
        b = i % NBUF
        j = i + NBUF - 1
        if j < n:
            jb = j % NBUF
            if sd[jb] is not None:
                sd[jb].wait()
            tj, cj = items[j]
            gd[jb] = pltpu.async_copy(tables[tj].at[idx_v.at[cj]], bufs[jb], gsems[jb])
        gd[b].wait()
        t, c = items[i]
        sd[b] = pltpu.async_copy(bufs[b], out_hbm.at[t, pl.ds(base + c * CHUNK, CHUNK)], ssems[b])
    for b in range(NBUF):
        if sd[b] is not None:
            sd[b].wait()


def kernel(cos_cached, sin_cached, position_ids):
    idx = position_ids.reshape(4, 64, CHUNK)
    out = _rope_gather(cos_cached, sin_cached, idx)
    return out.reshape(2, 4, 8192, DIM)
